# Initial kernel scaffold; baseline (speedup 1.0000x reference)
import jax
import jax.numpy as jnp
from jax.experimental import pallas as pl


def kernel(x, edge_index, batch, W1, b1, W2, b2, W3, b3, W4, b4):
    raise NotImplementedError("write your pallas kernel here")


# speedup vs baseline: 37.7979x; 37.7979x over previous
import functools

import jax
import jax.numpy as jnp
from jax import lax
from jax.experimental import pallas as pl
from jax.experimental.pallas import tpu as pltpu
from jax.experimental.pallas import tpu_sc as plsc

NC = 2
NS = 16
NW = NC * NS
LANES = 16
EC = 128
JC = 16
CHUNK = EC * JC

G = 64


def _mesh():
    return plsc.VectorSubcoreMesh(core_axis_name="c", subcore_axis_name="s")


_SC_PARAMS = pltpu.CompilerParams(use_tc_tiling_on_sc=False)


@functools.lru_cache(maxsize=None)
def _make_deg(nr: int, k_chunks: int):
    @functools.partial(
        pl.kernel,
        out_type=jax.ShapeDtypeStruct((NC, nr), jnp.float32),
        mesh=_mesh(),
        compiler_params=_SC_PARAMS,
        scratch_types=[
            pltpu.VMEM((JC, EC), jnp.int32),
            pltpu.VMEM((EC,), jnp.float32),
            pltpu.VMEM_SHARED((nr,), jnp.float32),
            pltpu.SemaphoreType.DMA,
        ],
    )
    def deg_kernel(dst_hbm, zeros_hbm, out_hbm, dst_v, ones_v, acc, sem):
        c = lax.axis_index("c")
        s = lax.axis_index("s")
        wid = c * NS + s
        zr = nr // NS
        pltpu.sync_copy(zeros_hbm.at[pl.ds(s * zr, zr)], acc.at[pl.ds(s * zr, zr)])
        for j in range(EC // LANES):
            ones_v[pl.ds(j * LANES, LANES)] = jnp.full((LANES,), 1.0, jnp.float32)
        plsc.subcore_barrier()

        def body(kk, _):
            row0 = (wid * k_chunks + kk) * JC
            pltpu.sync_copy(dst_hbm.at[pl.ds(row0, JC)], dst_v)
            descs = [
                pltpu.async_copy(ones_v, acc.at[dst_v.at[j]], sem, add=True)
                for j in range(JC)
            ]
            for d in descs:
                d.wait()
            return 0

        lax.fori_loop(0, k_chunks, body, 0)
        plsc.subcore_barrier()
        pltpu.sync_copy(acc.at[pl.ds(s * zr, zr)], out_hbm.at[c].at[pl.ds(s * zr, zr)])

    return deg_kernel


@functools.lru_cache(maxsize=None)
def _make_prop(nr: int, d: int, k_chunks: int):
    @functools.partial(
        pl.kernel,
        out_type=jax.ShapeDtypeStruct((NC, nr, d), jnp.float32),
        mesh=_mesh(),
        compiler_params=_SC_PARAMS,
        scratch_types=[
            pltpu.VMEM((JC, EC), jnp.int32),
            pltpu.VMEM((JC, EC), jnp.int32),
            pltpu.VMEM((CHUNK, d), jnp.float32),
            pltpu.VMEM_SHARED((nr, d), jnp.float32),
            pltpu.SemaphoreType.DMA,
            pltpu.SemaphoreType.DMA,
        ],
    )
    def prop_kernel(u_hbm, src_hbm, dst_hbm, zeros_hbm, out_hbm,
                    src_v, dst_v, rows_v, acc, gsem, ssem):
        c = lax.axis_index("c")
        s = lax.axis_index("s")
        wid = c * NS + s
        zr = nr // NS
        pltpu.sync_copy(zeros_hbm.at[pl.ds(s * zr, zr)], acc.at[pl.ds(s * zr, zr)])
        plsc.subcore_barrier()

        def body(kk, _):
            row0 = (wid * k_chunks + kk) * JC
            pltpu.sync_copy(src_hbm.at[pl.ds(row0, JC)], src_v)
            pltpu.sync_copy(dst_hbm.at[pl.ds(row0, JC)], dst_v)
            gd = [
                pltpu.async_copy(u_hbm.at[src_v.at[j]],
                                 rows_v.at[pl.ds(j * EC, EC)], gsem)
                for j in range(JC)
            ]
            for dd in gd:
                dd.wait()
            sd = [
                pltpu.async_copy(rows_v.at[pl.ds(j * EC, EC)],
                                 acc.at[dst_v.at[j]], ssem, add=True)
                for j in range(JC)
            ]
            for dd in sd:
                dd.wait()
            return 0

        lax.fori_loop(0, k_chunks, body, 0)
        plsc.subcore_barrier()
        pltpu.sync_copy(acc.at[pl.ds(s * zr, zr)], out_hbm.at[c].at[pl.ds(s * zr, zr)])

    return prop_kernel


@functools.lru_cache(maxsize=None)
def _make_prop_split(nr: int, k2: int):
    @functools.partial(
        pl.kernel,
        out_type=jax.ShapeDtypeStruct((NC, nr, 8), jnp.float32),
        mesh=_mesh(),
        compiler_params=_SC_PARAMS,
        scratch_types=[
            pltpu.VMEM((JC, EC), jnp.int32),
            pltpu.VMEM((JC, EC), jnp.int32),
            pltpu.VMEM((CHUNK, 8), jnp.float32),
            pltpu.VMEM_SHARED((nr, 8), jnp.float32),
            pltpu.SemaphoreType.DMA,
            pltpu.SemaphoreType.DMA,
        ],
    )
    def prop_kernel(u_hbm, src_hbm, dst_hbm, zeros_hbm, out_hbm,
                    src_v, dst_v, rows_v, acc, gsem, ssem):
        c = lax.axis_index("c")
        s = lax.axis_index("s")
        zr = nr // NS
        pltpu.sync_copy(zeros_hbm.at[pl.ds(s * zr, zr)], acc.at[pl.ds(s * zr, zr)])
        plsc.subcore_barrier()

        def body(kk, _):
            row0 = (s * k2 + kk) * JC
            pltpu.sync_copy(src_hbm.at[pl.ds(row0, JC)], src_v)
            pltpu.sync_copy(dst_hbm.at[pl.ds(row0, JC)], dst_v)
            gd = [
                pltpu.async_copy(u_hbm.at[c].at[src_v.at[j]],
                                 rows_v.at[pl.ds(j * EC, EC)], gsem)
                for j in range(JC)
            ]
            for dd in gd:
                dd.wait()
            sd = [
                pltpu.async_copy(rows_v.at[pl.ds(j * EC, EC)],
                                 acc.at[dst_v.at[j]], ssem, add=True)
                for j in range(JC)
            ]
            for dd in sd:
                dd.wait()
            return 0

        lax.fori_loop(0, k2, body, 0)
        plsc.subcore_barrier()
        pltpu.sync_copy(acc.at[pl.ds(s * zr, zr)], out_hbm.at[c].at[pl.ds(s * zr, zr)])

    return prop_kernel


@functools.lru_cache(maxsize=None)
def _make_pool(nr: int, acc_rows: int):
    k_chunks = nr // EC // NW

    @functools.partial(
        pl.kernel,
        out_type=jax.ShapeDtypeStruct((NC, acc_rows, 8), jnp.float32),
        mesh=_mesh(),
        compiler_params=_SC_PARAMS,
        scratch_types=[
            pltpu.VMEM((1, EC), jnp.int32),
            pltpu.VMEM((EC, 8), jnp.float32),
            pltpu.VMEM_SHARED((acc_rows, 8), jnp.float32),
            pltpu.SemaphoreType.DMA,
        ],
    )
    def pool_kernel(h_hbm, batch_hbm, zeros_hbm, out_hbm, idx_v, rows_v, acc, sem):
        c = lax.axis_index("c")
        s = lax.axis_index("s")
        wid = c * NS + s

        @pl.when(s == 0)
        def _():
            pltpu.sync_copy(zeros_hbm.at[pl.ds(0, acc_rows)], acc)

        plsc.subcore_barrier()

        def body(kk, _):
            r = wid * k_chunks + kk
            pltpu.sync_copy(batch_hbm.at[pl.ds(r, 1)], idx_v)
            pltpu.sync_copy(h_hbm.at[pl.ds(r * EC, EC)], rows_v)
            pltpu.async_copy(rows_v, acc.at[idx_v.at[0]], sem, add=True).wait()
            return 0

        lax.fori_loop(0, k_chunks, body, 0)
        plsc.subcore_barrier()

        @pl.when(s == 0)
        def _():
            pltpu.sync_copy(acc, out_hbm.at[c])

    return pool_kernel


_RB = 6400


def _row_spec(d):
    return pl.BlockSpec((_RB, d), lambda i: (i, 0))


def _pair_spec(d):
    return pl.BlockSpec((NC, _RB, d), lambda i: (0, i, 0))


def _full_spec(shape):
    return pl.BlockSpec(shape, lambda i: tuple(0 for _ in shape))


def _t0(degp, x_pad, nr):
    def body(dp_ref, x_ref, dinv_ref, u1_ref):
        deg = dp_ref[0] + dp_ref[1] + 1.0
        dinv = lax.rsqrt(deg)
        dinv_ref[...] = dinv
        u1_ref[...] = x_ref[...] * dinv

    return pl.pallas_call(
        body,
        grid=(nr // _RB,),
        in_specs=[_pair_spec(1), _row_spec(8)],
        out_specs=[_row_spec(1), _row_spec(8)],
        out_shape=[
            jax.ShapeDtypeStruct((nr, 1), jnp.float32),
            jax.ShapeDtypeStruct((nr, 8), jnp.float32),
        ],
    )(degp, x_pad)


def _t1(P1, u1, dinv, W1p, b1, nr):
    def body(p_ref, u_ref, dinv_ref, w_ref, b_ref, out_ref):
        t = dinv_ref[...] * (p_ref[0] + p_ref[1] + u_ref[...])
        h = jnp.maximum(
            jnp.dot(t, w_ref[...], preferred_element_type=jnp.float32)
            + b_ref[...], 0.0)
        hu = dinv_ref[...] * h
        out_ref[0, :, :] = hu[:, :8]
        out_ref[1, :, :] = hu[:, 8:]

    return pl.pallas_call(
        body,
        grid=(nr // _RB,),
        in_specs=[_pair_spec(8), _row_spec(8), _row_spec(1),
                  _full_spec((8, 16)), _full_spec((1, 16))],
        out_specs=_pair_spec(8),
        out_shape=jax.ShapeDtypeStruct((NC, nr, 8), jnp.float32),
    )(P1, u1, dinv, W1p, b1)


def _t2(P2, u2, dinv, W2, b2, W3, nr):
    def body(p_ref, u_ref, dinv_ref, w2_ref, b2_ref, w3_ref, out_ref):
        pfull = jnp.concatenate([p_ref[0], p_ref[1]], axis=1)
        ufull = jnp.concatenate([u_ref[0], u_ref[1]], axis=1)
        t = dinv_ref[...] * (pfull + ufull)
        h = jnp.maximum(
            jnp.dot(t, w2_ref[...], preferred_element_type=jnp.float32)
            + b2_ref[...], 0.0)
        v = jnp.dot(h, w3_ref[...], preferred_element_type=jnp.float32)
        hu = dinv_ref[...] * v
        out_ref[0, :, :] = hu[:, :8]
        out_ref[1, :, :] = hu[:, 8:]

    return pl.pallas_call(
        body,
        grid=(nr // _RB,),
        in_specs=[_pair_spec(8), _pair_spec(8), _row_spec(1),
                  _full_spec((16, 32)), _full_spec((1, 32)), _full_spec((32, 16))],
        out_specs=_pair_spec(8),
        out_shape=jax.ShapeDtypeStruct((NC, nr, 8), jnp.float32),
    )(P2, u2, dinv, W2, b2, W3)


def _t3(P3, u3, dinv, b3, W4p, nr):
    def body(p_ref, u_ref, dinv_ref, b3_ref, w4_ref, out_ref):
        pfull = jnp.concatenate([p_ref[0], p_ref[1]], axis=1)
        ufull = jnp.concatenate([u_ref[0], u_ref[1]], axis=1)
        h = jnp.maximum(
            dinv_ref[...] * (pfull + ufull) + b3_ref[...], 0.0)
        v = jnp.dot(h, w4_ref[...], preferred_element_type=jnp.float32)
        out_ref[...] = dinv_ref[...] * v

    return pl.pallas_call(
        body,
        grid=(nr // _RB,),
        in_specs=[_pair_spec(8), _pair_spec(8), _row_spec(1),
                  _full_spec((1, 16)), _full_spec((16, 8))],
        out_specs=_row_spec(8),
        out_shape=jax.ShapeDtypeStruct((nr, 8), jnp.float32),
    )(P3, u3, dinv, b3, W4p)


def _t4(P4, u4, dinv, b4, nr, n):
    def body(p_ref, u_ref, dinv_ref, b4_ref, out_ref):
        i = pl.program_id(0)
        h4 = dinv_ref[...] * (p_ref[0] + p_ref[1] + u_ref[...]) + b4_ref[...]
        rows = lax.broadcasted_iota(jnp.int32, (_RB, 1), 0) + i * _RB
        ones = jnp.where(rows < n, 1.0, 0.0).astype(jnp.float32)
        out_ref[...] = jnp.concatenate([h4[:, :2], ones, h4[:, 3:]], axis=1)

    return pl.pallas_call(
        body,
        grid=(nr // _RB,),
        in_specs=[_pair_spec(8), _row_spec(8), _row_spec(1), _full_spec((1, 8))],
        out_specs=_row_spec(8),
        out_shape=jax.ShapeDtypeStruct((nr, 8), jnp.float32),
    )(P4, u4, dinv, b4)


def _t5(pools, acc_rows):
    def body(p_ref, out_ref):
        s = p_ref[0] + p_ref[1]
        sums = s[:G, :2]
        cnt = jnp.maximum(s[:G, 2:3], 1.0)
        mean = sums / cnt
        m = jnp.max(mean, axis=1, keepdims=True)
        e = jnp.exp(mean - m)
        lse = m + jnp.log(jnp.sum(e, axis=1, keepdims=True))
        out_ref[...] = mean - lse

    return pl.pallas_call(
        body,
        grid=(1,),
        in_specs=[pl.BlockSpec((NC, acc_rows, 8), lambda i: (0, 0, 0))],
        out_specs=pl.BlockSpec((G, 2), lambda i: (0, 0)),
        out_shape=jax.ShapeDtypeStruct((G, 2), jnp.float32),
    )(pools)


def kernel(x, edge_index, batch, W1, b1, W2, b2, W3, b3, W4, b4):
    n = x.shape[0]
    e = edge_index.shape[1]
    blk = NW * EC
    nr = ((n + 1 + blk - 1) // blk) * blk

    k_chunks = -(-e // (NW * CHUNK))
    e_pad = NW * k_chunks * CHUNK
    pad_e = e_pad - e
    padv = jnp.full((pad_e,), n, jnp.int32)
    srcp = jnp.concatenate([edge_index[0], padv]).reshape(e_pad // EC, EC)
    dstp = jnp.concatenate([edge_index[1], padv]).reshape(e_pad // EC, EC)
    batchp = jnp.concatenate(
        [batch, jnp.full((nr - n,), G, jnp.int32)]).reshape(nr // EC, EC)

    x_pad = jnp.pad(x, ((0, nr - n), (0, 8 - x.shape[1])))
    W1p = jnp.pad(W1, ((0, 8 - W1.shape[0]), (0, 0)))
    W4p = jnp.pad(W4, ((0, 0), (0, 6)))
    b1r = b1.reshape(1, -1)
    b2r = b2.reshape(1, -1)
    b3r = b3.reshape(1, -1)
    b4r = jnp.pad(b4.reshape(1, -1), ((0, 0), (0, 6)))

    z8 = jnp.zeros((nr, 8), jnp.float32)
    z1 = jnp.zeros((nr,), jnp.float32)

    degp = _make_deg(nr, k_chunks)(dstp, z1)
    dinv, u1 = _t0(degp.reshape(NC, nr, 1), x_pad, nr)

    P1 = _make_prop(nr, 8, k_chunks)(u1, srcp, dstp, z8)
    u2 = _t1(P1, u1, dinv, W1p, b1r, nr)

    k2 = NC * k_chunks
    P2 = _make_prop_split(nr, k2)(u2, srcp, dstp, z8)
    u3 = _t2(P2, u2, dinv, W2, b2r, W3, nr)

    P3 = _make_prop_split(nr, k2)(u3, srcp, dstp, z8)
    u4 = _t3(P3, u3, dinv, b3r, W4p, nr)

    P4 = _make_prop(nr, 8, k_chunks)(u4, srcp, dstp, z8)
    h4aug = _t4(P4, u4, dinv, b4r, nr, n)

    acc_rows = 72
    pools = _make_pool(nr, acc_rows)(h4aug, batchp, z8)
    return _t5(pools, acc_rows)


# speedup vs baseline: 43.8179x; 1.1593x over previous
import functools

import jax
import jax.numpy as jnp
from jax import lax
from jax.experimental import pallas as pl
from jax.experimental.pallas import tpu as pltpu
from jax.experimental.pallas import tpu_sc as plsc

NC = 2
NS = 16
NW = NC * NS
LANES = 16
EC = 128
JC = 16
CHUNK = EC * JC

G = 64


def _mesh():
    return plsc.VectorSubcoreMesh(core_axis_name="c", subcore_axis_name="s")


_SC_PARAMS = pltpu.CompilerParams(use_tc_tiling_on_sc=False)


@functools.lru_cache(maxsize=None)
def _make_deg(nr: int, k_chunks: int):
    @functools.partial(
        pl.kernel,
        out_type=jax.ShapeDtypeStruct((NC, nr), jnp.float32),
        mesh=_mesh(),
        compiler_params=_SC_PARAMS,
        scratch_types=[
            pltpu.VMEM((JC, EC), jnp.int32),
            pltpu.VMEM((EC,), jnp.float32),
            pltpu.VMEM_SHARED((nr,), jnp.float32),
            pltpu.SemaphoreType.DMA,
        ],
    )
    def deg_kernel(dst_hbm, zeros_hbm, out_hbm, dst_v, ones_v, acc, sem):
        c = lax.axis_index("c")
        s = lax.axis_index("s")
        wid = c * NS + s
        zr = nr // NS
        pltpu.sync_copy(zeros_hbm.at[pl.ds(s * zr, zr)], acc.at[pl.ds(s * zr, zr)])
        for j in range(EC // LANES):
            ones_v[pl.ds(j * LANES, LANES)] = jnp.full((LANES,), 1.0, jnp.float32)
        plsc.subcore_barrier()

        def body(kk, _):
            row0 = (wid * k_chunks + kk) * JC
            pltpu.sync_copy(dst_hbm.at[pl.ds(row0, JC)], dst_v)
            descs = [
                pltpu.async_copy(ones_v, acc.at[dst_v.at[j]], sem, add=True)
                for j in range(JC)
            ]
            for d in descs:
                d.wait()
            return 0

        lax.fori_loop(0, k_chunks, body, 0)
        plsc.subcore_barrier()
        pltpu.sync_copy(acc.at[pl.ds(s * zr, zr)], out_hbm.at[c].at[pl.ds(s * zr, zr)])

    return deg_kernel


@functools.lru_cache(maxsize=None)
def _make_prop(nr: int, d: int, rows_pw: int, jc: int):
    n_chunks = rows_pw // jc

    @functools.partial(
        pl.kernel,
        out_type=jax.ShapeDtypeStruct((NC, nr, d), jnp.float32),
        mesh=_mesh(),
        compiler_params=_SC_PARAMS,
        scratch_types=[
            pltpu.VMEM((jc, EC), jnp.int32),
            pltpu.VMEM((jc, EC), jnp.int32),
            pltpu.VMEM((jc * EC, d), jnp.float32),
            pltpu.VMEM_SHARED((nr, d), jnp.float32),
            pltpu.SemaphoreType.DMA,
            pltpu.SemaphoreType.DMA,
        ],
    )
    def prop_kernel(u_hbm, src_hbm, dst_hbm, zeros_hbm, out_hbm,
                    src_v, dst_v, rows_v, acc, gsem, ssem):
        c = lax.axis_index("c")
        s = lax.axis_index("s")
        wid = c * NS + s
        zr = nr // NS
        pltpu.sync_copy(zeros_hbm.at[pl.ds(s * zr, zr)], acc.at[pl.ds(s * zr, zr)])
        plsc.subcore_barrier()

        def body(kk, _):
            row0 = wid * rows_pw + kk * jc
            pltpu.sync_copy(src_hbm.at[pl.ds(row0, jc)], src_v)
            pltpu.sync_copy(dst_hbm.at[pl.ds(row0, jc)], dst_v)
            gd = [
                pltpu.async_copy(u_hbm.at[src_v.at[j]],
                                 rows_v.at[pl.ds(j * EC, EC)], gsem)
                for j in range(jc)
            ]
            for dd in gd:
                dd.wait()
            sd = [
                pltpu.async_copy(rows_v.at[pl.ds(j * EC, EC)],
                                 acc.at[dst_v.at[j]], ssem, add=True)
                for j in range(jc)
            ]
            for dd in sd:
                dd.wait()
            return 0

        lax.fori_loop(0, n_chunks, body, 0)
        plsc.subcore_barrier()
        pltpu.sync_copy(acc.at[pl.ds(s * zr, zr)], out_hbm.at[c].at[pl.ds(s * zr, zr)])

    return prop_kernel


@functools.lru_cache(maxsize=None)
def _make_pool(nr: int, acc_rows: int):
    k_chunks = nr // EC // NW

    @functools.partial(
        pl.kernel,
        out_type=jax.ShapeDtypeStruct((NC, acc_rows, 8), jnp.float32),
        mesh=_mesh(),
        compiler_params=_SC_PARAMS,
        scratch_types=[
            pltpu.VMEM((1, EC), jnp.int32),
            pltpu.VMEM((EC, 8), jnp.float32),
            pltpu.VMEM_SHARED((acc_rows, 8), jnp.float32),
            pltpu.SemaphoreType.DMA,
        ],
    )
    def pool_kernel(h_hbm, batch_hbm, zeros_hbm, out_hbm, idx_v, rows_v, acc, sem):
        c = lax.axis_index("c")
        s = lax.axis_index("s")
        wid = c * NS + s

        @pl.when(s == 0)
        def _():
            pltpu.sync_copy(zeros_hbm.at[pl.ds(0, acc_rows)], acc)

        plsc.subcore_barrier()

        def body(kk, _):
            r = wid * k_chunks + kk
            pltpu.sync_copy(batch_hbm.at[pl.ds(r, 1)], idx_v)
            pltpu.sync_copy(h_hbm.at[pl.ds(r * EC, EC)], rows_v)
            pltpu.async_copy(rows_v, acc.at[idx_v.at[0]], sem, add=True).wait()
            return 0

        lax.fori_loop(0, k_chunks, body, 0)
        plsc.subcore_barrier()

        @pl.when(s == 0)
        def _():
            pltpu.sync_copy(acc, out_hbm.at[c])

    return pool_kernel


_RB = 6400


def _row_spec(d):
    return pl.BlockSpec((_RB, d), lambda i: (i, 0))


def _pair_spec(d):
    return pl.BlockSpec((NC, _RB, d), lambda i: (0, i, 0))


def _full_spec(shape):
    return pl.BlockSpec(shape, lambda i: tuple(0 for _ in shape))


def _t0(degp, x_pad, nr):
    def body(dp_ref, x_ref, dinv_ref, u1_ref):
        deg = dp_ref[0] + dp_ref[1] + 1.0
        dinv = lax.rsqrt(deg)
        dinv_ref[...] = dinv
        u1_ref[...] = x_ref[...] * dinv

    return pl.pallas_call(
        body,
        grid=(nr // _RB,),
        in_specs=[_pair_spec(1), _row_spec(8)],
        out_specs=[_row_spec(1), _row_spec(8)],
        out_shape=[
            jax.ShapeDtypeStruct((nr, 1), jnp.float32),
            jax.ShapeDtypeStruct((nr, 8), jnp.float32),
        ],
    )(degp, x_pad)


def _t1(P1, u1, dinv, W1p, b1, nr):
    def body(p_ref, u_ref, dinv_ref, w_ref, b_ref, out_ref):
        t = dinv_ref[...] * (p_ref[0] + p_ref[1] + u_ref[...])
        h = jnp.maximum(
            jnp.dot(t, w_ref[...], preferred_element_type=jnp.float32)
            + b_ref[...], 0.0)
        out_ref[...] = dinv_ref[...] * h

    return pl.pallas_call(
        body,
        grid=(nr // _RB,),
        in_specs=[_pair_spec(8), _row_spec(8), _row_spec(1),
                  _full_spec((8, 16)), _full_spec((1, 16))],
        out_specs=_row_spec(16),
        out_shape=jax.ShapeDtypeStruct((nr, 16), jnp.float32),
    )(P1, u1, dinv, W1p, b1)


def _t2(P2, u2, dinv, W2, b2, W3, nr):
    def body(p_ref, u_ref, dinv_ref, w2_ref, b2_ref, w3_ref, out_ref):
        t = dinv_ref[...] * (p_ref[0] + p_ref[1] + u_ref[...])
        h = jnp.maximum(
            jnp.dot(t, w2_ref[...], preferred_element_type=jnp.float32)
            + b2_ref[...], 0.0)
        v = jnp.dot(h, w3_ref[...], preferred_element_type=jnp.float32)
        out_ref[...] = dinv_ref[...] * v

    return pl.pallas_call(
        body,
        grid=(nr // _RB,),
        in_specs=[_pair_spec(16), _row_spec(16), _row_spec(1),
                  _full_spec((16, 32)), _full_spec((1, 32)), _full_spec((32, 16))],
        out_specs=_row_spec(16),
        out_shape=jax.ShapeDtypeStruct((nr, 16), jnp.float32),
    )(P2, u2, dinv, W2, b2, W3)


def _t3(P3, u3, dinv, b3, W4p, nr):
    def body(p_ref, u_ref, dinv_ref, b3_ref, w4_ref, out_ref):
        h = jnp.maximum(
            dinv_ref[...] * (p_ref[0] + p_ref[1] + u_ref[...]) + b3_ref[...],
            0.0)
        v = jnp.dot(h, w4_ref[...], preferred_element_type=jnp.float32)
        out_ref[...] = dinv_ref[...] * v

    return pl.pallas_call(
        body,
        grid=(nr // _RB,),
        in_specs=[_pair_spec(16), _row_spec(16), _row_spec(1),
                  _full_spec((1, 16)), _full_spec((16, 8))],
        out_specs=_row_spec(8),
        out_shape=jax.ShapeDtypeStruct((nr, 8), jnp.float32),
    )(P3, u3, dinv, b3, W4p)


def _t4(P4, u4, dinv, b4, nr, n):
    def body(p_ref, u_ref, dinv_ref, b4_ref, out_ref):
        i = pl.program_id(0)
        h4 = dinv_ref[...] * (p_ref[0] + p_ref[1] + u_ref[...]) + b4_ref[...]
        rows = lax.broadcasted_iota(jnp.int32, (_RB, 1), 0) + i * _RB
        ones = jnp.where(rows < n, 1.0, 0.0).astype(jnp.float32)
        out_ref[...] = jnp.concatenate([h4[:, :2], ones, h4[:, 3:]], axis=1)

    return pl.pallas_call(
        body,
        grid=(nr // _RB,),
        in_specs=[_pair_spec(8), _row_spec(8), _row_spec(1), _full_spec((1, 8))],
        out_specs=_row_spec(8),
        out_shape=jax.ShapeDtypeStruct((nr, 8), jnp.float32),
    )(P4, u4, dinv, b4)


def _t5(pools, acc_rows):
    def body(p_ref, out_ref):
        s = p_ref[0] + p_ref[1]
        sums = s[:G, :2]
        cnt = jnp.maximum(s[:G, 2:3], 1.0)
        mean = sums / cnt
        m = jnp.max(mean, axis=1, keepdims=True)
        e = jnp.exp(mean - m)
        lse = m + jnp.log(jnp.sum(e, axis=1, keepdims=True))
        out_ref[...] = mean - lse

    return pl.pallas_call(
        body,
        grid=(1,),
        in_specs=[pl.BlockSpec((NC, acc_rows, 8), lambda i: (0, 0, 0))],
        out_specs=pl.BlockSpec((G, 2), lambda i: (0, 0)),
        out_shape=jax.ShapeDtypeStruct((G, 2), jnp.float32),
    )(pools)


def kernel(x, edge_index, batch, W1, b1, W2, b2, W3, b3, W4, b4):
    n = x.shape[0]
    e = edge_index.shape[1]
    blk = NW * EC
    nr = ((n + 1 + blk - 1) // blk) * blk

    k_chunks = -(-e // (NW * CHUNK))
    e_pad = NW * k_chunks * CHUNK
    pad_e = e_pad - e
    padv = jnp.full((pad_e,), n, jnp.int32)
    srcp = jnp.concatenate([edge_index[0], padv]).reshape(e_pad // EC, EC)
    dstp = jnp.concatenate([edge_index[1], padv]).reshape(e_pad // EC, EC)
    batchp = jnp.concatenate(
        [batch, jnp.full((nr - n,), G, jnp.int32)]).reshape(nr // EC, EC)

    x_pad = jnp.pad(x, ((0, nr - n), (0, 8 - x.shape[1])))
    W1p = jnp.pad(W1, ((0, 8 - W1.shape[0]), (0, 0)))
    W4p = jnp.pad(W4, ((0, 0), (0, 6)))
    b1r = b1.reshape(1, -1)
    b2r = b2.reshape(1, -1)
    b3r = b3.reshape(1, -1)
    b4r = jnp.pad(b4.reshape(1, -1), ((0, 0), (0, 6)))

    z16 = jnp.zeros((nr, 16), jnp.float32)
    z8 = jnp.zeros((nr, 8), jnp.float32)
    z1 = jnp.zeros((nr,), jnp.float32)

    degp = _make_deg(nr, k_chunks)(dstp, z1)
    dinv, u1 = _t0(degp.reshape(NC, nr, 1), x_pad, nr)

    rows_pw = k_chunks * JC
    P1 = _make_prop(nr, 8, rows_pw, 16)(u1, srcp, dstp, z8)
    u2 = _t1(P1, u1, dinv, W1p, b1r, nr)

    P2 = _make_prop(nr, 16, rows_pw, 8)(u2, srcp, dstp, z16)
    u3 = _t2(P2, u2, dinv, W2, b2r, W3, nr)

    P3 = _make_prop(nr, 16, rows_pw, 8)(u3, srcp, dstp, z16)
    u4 = _t3(P3, u3, dinv, b3r, W4p, nr)

    P4 = _make_prop(nr, 8, rows_pw, 16)(u4, srcp, dstp, z8)
    h4aug = _t4(P4, u4, dinv, b4r, nr, n)

    acc_rows = 72
    pools = _make_pool(nr, acc_rows)(h4aug, batchp, z8)
    return _t5(pools, acc_rows)


# speedup vs baseline: 45.7683x; 1.0445x over previous
import functools

import jax
import jax.numpy as jnp
from jax import lax
from jax.experimental import pallas as pl
from jax.experimental.pallas import tpu as pltpu
from jax.experimental.pallas import tpu_sc as plsc

NC = 2
NS = 16
NW = NC * NS
LANES = 16
EC = 128
JC = 16
CHUNK = EC * JC

G = 64


def _mesh():
    return plsc.VectorSubcoreMesh(core_axis_name="c", subcore_axis_name="s")


_SC_PARAMS = pltpu.CompilerParams(use_tc_tiling_on_sc=False)


@functools.lru_cache(maxsize=None)
def _make_deg(nr: int, k_chunks: int):
    @functools.partial(
        pl.kernel,
        out_type=jax.ShapeDtypeStruct((NC, nr), jnp.float32),
        mesh=_mesh(),
        compiler_params=_SC_PARAMS,
        scratch_types=[
            pltpu.VMEM((JC, EC), jnp.int32),
            pltpu.VMEM((EC,), jnp.float32),
            pltpu.VMEM_SHARED((nr,), jnp.float32),
            pltpu.SemaphoreType.DMA,
        ],
    )
    def deg_kernel(dst_hbm, zeros_hbm, out_hbm, dst_v, ones_v, acc, sem):
        c = lax.axis_index("c")
        s = lax.axis_index("s")
        wid = c * NS + s
        zr = nr // NS
        pltpu.sync_copy(zeros_hbm.at[pl.ds(s * zr, zr)], acc.at[pl.ds(s * zr, zr)])
        for j in range(EC // LANES):
            ones_v[pl.ds(j * LANES, LANES)] = jnp.full((LANES,), 1.0, jnp.float32)
        plsc.subcore_barrier()

        def body(kk, _):
            row0 = (wid * k_chunks + kk) * JC
            pltpu.sync_copy(dst_hbm.at[pl.ds(row0, JC)], dst_v)
            descs = [
                pltpu.async_copy(ones_v, acc.at[dst_v.at[j]], sem, add=True)
                for j in range(JC)
            ]
            for d in descs:
                d.wait()
            return 0

        lax.fori_loop(0, k_chunks, body, 0)
        plsc.subcore_barrier()
        pltpu.sync_copy(acc.at[pl.ds(s * zr, zr)], out_hbm.at[c].at[pl.ds(s * zr, zr)])

    return deg_kernel


@functools.lru_cache(maxsize=None)
def _make_prop(nr: int, d: int, rows_pw: int, jc: int):
    n_chunks = rows_pw // jc

    @functools.partial(
        pl.kernel,
        out_type=jax.ShapeDtypeStruct((NC, nr, d), jnp.float32),
        mesh=_mesh(),
        compiler_params=_SC_PARAMS,
        scratch_types=[
            pltpu.VMEM((jc, EC), jnp.int32),
            pltpu.VMEM((jc, EC), jnp.int32),
            pltpu.VMEM((jc * EC, d), jnp.float32),
            pltpu.VMEM_SHARED((nr, d), jnp.float32),
            pltpu.SemaphoreType.DMA,
            pltpu.SemaphoreType.DMA,
        ],
    )
    def prop_kernel(u_hbm, src_hbm, dst_hbm, zeros_hbm, out_hbm,
                    src_v, dst_v, rows_v, acc, gsem, ssem):
        c = lax.axis_index("c")
        s = lax.axis_index("s")
        wid = c * NS + s
        zr = nr // NS
        pltpu.sync_copy(zeros_hbm.at[pl.ds(s * zr, zr)], acc.at[pl.ds(s * zr, zr)])
        plsc.subcore_barrier()

        def body(kk, _):
            row0 = wid * rows_pw + kk * jc
            pltpu.sync_copy(src_hbm.at[pl.ds(row0, jc)], src_v)
            pltpu.sync_copy(dst_hbm.at[pl.ds(row0, jc)], dst_v)
            gd = [
                pltpu.async_copy(u_hbm.at[src_v.at[j]],
                                 rows_v.at[pl.ds(j * EC, EC)], gsem)
                for j in range(jc)
            ]
            for dd in gd:
                dd.wait()
            sd = [
                pltpu.async_copy(rows_v.at[pl.ds(j * EC, EC)],
                                 acc.at[dst_v.at[j]], ssem, add=True)
                for j in range(jc)
            ]
            for dd in sd:
                dd.wait()
            return 0

        lax.fori_loop(0, n_chunks, body, 0)
        plsc.subcore_barrier()
        pltpu.sync_copy(acc.at[pl.ds(s * zr, zr)], out_hbm.at[c].at[pl.ds(s * zr, zr)])

    return prop_kernel


@functools.lru_cache(maxsize=None)
def _make_pool(nr: int, acc_rows: int):
    k_chunks = nr // EC // NW

    @functools.partial(
        pl.kernel,
        out_type=jax.ShapeDtypeStruct((NC, acc_rows, 8), jnp.float32),
        mesh=_mesh(),
        compiler_params=_SC_PARAMS,
        scratch_types=[
            pltpu.VMEM((1, EC), jnp.int32),
            pltpu.VMEM((EC, 8), jnp.float32),
            pltpu.VMEM_SHARED((acc_rows, 8), jnp.float32),
            pltpu.SemaphoreType.DMA,
        ],
    )
    def pool_kernel(h_hbm, batch_hbm, zeros_hbm, out_hbm, idx_v, rows_v, acc, sem):
        c = lax.axis_index("c")
        s = lax.axis_index("s")
        wid = c * NS + s

        @pl.when(s == 0)
        def _():
            pltpu.sync_copy(zeros_hbm.at[pl.ds(0, acc_rows)], acc)

        plsc.subcore_barrier()

        def body(kk, _):
            r = wid * k_chunks + kk
            pltpu.sync_copy(batch_hbm.at[pl.ds(r, 1)], idx_v)
            pltpu.sync_copy(h_hbm.at[pl.ds(r * EC, EC)], rows_v)
            pltpu.async_copy(rows_v, acc.at[idx_v.at[0]], sem, add=True).wait()
            return 0

        lax.fori_loop(0, k_chunks, body, 0)
        plsc.subcore_barrier()

        @pl.when(s == 0)
        def _():
            pltpu.sync_copy(acc, out_hbm.at[c])

    return pool_kernel


_RB = 6400


def _row_spec(d):
    return pl.BlockSpec((_RB, d), lambda i: (i, 0))


def _pair_spec(d):
    return pl.BlockSpec((NC, _RB, d), lambda i: (0, i, 0))


def _full_spec(shape):
    return pl.BlockSpec(shape, lambda i: tuple(0 for _ in shape))


def _t0(degp, x_pad, nr):
    def body(dp_ref, x_ref, dinv_ref, u1_ref):
        deg = dp_ref[0] + dp_ref[1] + 1.0
        dinv = lax.rsqrt(deg).reshape(_RB, 1)
        dinv8 = jnp.broadcast_to(dinv, (_RB, 8))
        dinv_ref[...] = dinv8
        u1_ref[...] = x_ref[...] * dinv8

    return pl.pallas_call(
        body,
        grid=(nr // _RB,),
        in_specs=[pl.BlockSpec((NC, _RB), lambda i: (0, i)), _row_spec(8)],
        out_specs=[_row_spec(8), _row_spec(8)],
        out_shape=[
            jax.ShapeDtypeStruct((nr, 8), jnp.float32),
            jax.ShapeDtypeStruct((nr, 8), jnp.float32),
        ],
    )(degp, x_pad)


def _t1(P1, u1, dinv, W1p, b1, nr):
    def body(p_ref, u_ref, dinv_ref, w_ref, b_ref, out_ref):
        d8 = dinv_ref[...]
        t = d8 * (p_ref[0] + p_ref[1] + u_ref[...])
        h = jnp.maximum(
            jnp.dot(t, w_ref[...], preferred_element_type=jnp.float32)
            + b_ref[...], 0.0)
        out_ref[...] = jnp.concatenate([d8, d8], axis=1) * h

    return pl.pallas_call(
        body,
        grid=(nr // _RB,),
        in_specs=[_pair_spec(8), _row_spec(8), _row_spec(8),
                  _full_spec((8, 16)), _full_spec((1, 16))],
        out_specs=_row_spec(16),
        out_shape=jax.ShapeDtypeStruct((nr, 16), jnp.float32),
    )(P1, u1, dinv, W1p, b1)


def _t2(P2, u2, dinv, W2, b2, W3, nr):
    def body(p_ref, u_ref, dinv_ref, w2_ref, b2_ref, w3_ref, out_ref):
        d8 = dinv_ref[...]
        d16 = jnp.concatenate([d8, d8], axis=1)
        t = d16 * (p_ref[0] + p_ref[1] + u_ref[...])
        h = jnp.maximum(
            jnp.dot(t, w2_ref[...], preferred_element_type=jnp.float32)
            + b2_ref[...], 0.0)
        v = jnp.dot(h, w3_ref[...], preferred_element_type=jnp.float32)
        out_ref[...] = d16 * v

    return pl.pallas_call(
        body,
        grid=(nr // _RB,),
        in_specs=[_pair_spec(16), _row_spec(16), _row_spec(8),
                  _full_spec((16, 32)), _full_spec((1, 32)), _full_spec((32, 16))],
        out_specs=_row_spec(16),
        out_shape=jax.ShapeDtypeStruct((nr, 16), jnp.float32),
    )(P2, u2, dinv, W2, b2, W3)


def _t3(P3, u3, dinv, b3, W4p, nr):
    def body(p_ref, u_ref, dinv_ref, b3_ref, w4_ref, out_ref):
        d8 = dinv_ref[...]
        d16 = jnp.concatenate([d8, d8], axis=1)
        h = jnp.maximum(
            d16 * (p_ref[0] + p_ref[1] + u_ref[...]) + b3_ref[...], 0.0)
        v = jnp.dot(h, w4_ref[...], preferred_element_type=jnp.float32)
        out_ref[...] = d8 * v

    return pl.pallas_call(
        body,
        grid=(nr // _RB,),
        in_specs=[_pair_spec(16), _row_spec(16), _row_spec(8),
                  _full_spec((1, 16)), _full_spec((16, 8))],
        out_specs=_row_spec(8),
        out_shape=jax.ShapeDtypeStruct((nr, 8), jnp.float32),
    )(P3, u3, dinv, b3, W4p)


def _t4(P4, u4, dinv, b4, nr, n):
    def body(p_ref, u_ref, dinv_ref, b4_ref, out_ref):
        i = pl.program_id(0)
        h4 = dinv_ref[...] * (p_ref[0] + p_ref[1] + u_ref[...]) + b4_ref[...]
        rows = lax.broadcasted_iota(jnp.int32, (_RB, 1), 0) + i * _RB
        ones = jnp.where(rows < n, 1.0, 0.0).astype(jnp.float32)
        out_ref[...] = jnp.concatenate([h4[:, :2], ones, h4[:, 3:]], axis=1)

    return pl.pallas_call(
        body,
        grid=(nr // _RB,),
        in_specs=[_pair_spec(8), _row_spec(8), _row_spec(8), _full_spec((1, 8))],
        out_specs=_row_spec(8),
        out_shape=jax.ShapeDtypeStruct((nr, 8), jnp.float32),
    )(P4, u4, dinv, b4)


def _t5(pools, acc_rows):
    def body(p_ref, out_ref):
        s = p_ref[0] + p_ref[1]
        sums = s[:G, :2]
        cnt = jnp.maximum(s[:G, 2:3], 1.0)
        mean = sums / cnt
        m = jnp.max(mean, axis=1, keepdims=True)
        e = jnp.exp(mean - m)
        lse = m + jnp.log(jnp.sum(e, axis=1, keepdims=True))
        out_ref[...] = mean - lse

    return pl.pallas_call(
        body,
        grid=(1,),
        in_specs=[pl.BlockSpec((NC, acc_rows, 8), lambda i: (0, 0, 0))],
        out_specs=pl.BlockSpec((G, 2), lambda i: (0, 0)),
        out_shape=jax.ShapeDtypeStruct((G, 2), jnp.float32),
    )(pools)


def kernel(x, edge_index, batch, W1, b1, W2, b2, W3, b3, W4, b4):
    n = x.shape[0]
    e = edge_index.shape[1]
    blk = NW * EC
    nr = ((n + 1 + blk - 1) // blk) * blk

    k_chunks = -(-e // (NW * CHUNK))
    e_pad = NW * k_chunks * CHUNK
    pad_e = e_pad - e
    padv = jnp.full((pad_e,), n, jnp.int32)
    srcp = jnp.concatenate([edge_index[0], padv]).reshape(e_pad // EC, EC)
    dstp = jnp.concatenate([edge_index[1], padv]).reshape(e_pad // EC, EC)
    batchp = jnp.concatenate(
        [batch, jnp.full((nr - n,), G, jnp.int32)]).reshape(nr // EC, EC)

    x_pad = jnp.pad(x, ((0, nr - n), (0, 8 - x.shape[1])))
    W1p = jnp.pad(W1, ((0, 8 - W1.shape[0]), (0, 0)))
    W4p = jnp.pad(W4, ((0, 0), (0, 6)))
    b1r = b1.reshape(1, -1)
    b2r = b2.reshape(1, -1)
    b3r = b3.reshape(1, -1)
    b4r = jnp.pad(b4.reshape(1, -1), ((0, 0), (0, 6)))

    z16 = jnp.zeros((nr, 16), jnp.float32)
    z8 = jnp.zeros((nr, 8), jnp.float32)
    z1 = jnp.zeros((nr,), jnp.float32)

    degp = _make_deg(nr, k_chunks)(dstp, z1)
    dinv, u1 = _t0(degp, x_pad, nr)

    rows_pw = k_chunks * JC
    P1 = _make_prop(nr, 8, rows_pw, 16)(u1, srcp, dstp, z8)
    u2 = _t1(P1, u1, dinv, W1p, b1r, nr)

    P2 = _make_prop(nr, 16, rows_pw, 8)(u2, srcp, dstp, z16)
    u3 = _t2(P2, u2, dinv, W2, b2r, W3, nr)

    P3 = _make_prop(nr, 16, rows_pw, 8)(u3, srcp, dstp, z16)
    u4 = _t3(P3, u3, dinv, b3r, W4p, nr)

    P4 = _make_prop(nr, 8, rows_pw, 16)(u4, srcp, dstp, z8)
    h4aug = _t4(P4, u4, dinv, b4r, nr, n)

    acc_rows = 72
    pools = _make_pool(nr, acc_rows)(h4aug, batchp, z8)
    return _t5(pools, acc_rows)
